# baseline (speedup 1.0000x reference)
import functools

import jax
import jax.numpy as jnp
from jax import lax
from jax.experimental import pallas as pl
from jax.experimental.pallas import tpu as pltpu
from jax.experimental.pallas import tpu_sc as plsc

_EMBED = 32
_BATCH = 16384
_HB = _BATCH // 2
_NC = 2
_NS = 16
_NW = _NC * _NS
_BPW = _HB // _NW
_LROWS = (_HB // 128) * _EMBED


def _gather_body(user_hbm, item_hbm, utab_hbm, itab_hbm, uout_hbm, iout_hbm,
                 uidx_v, iidx_v, buf, stage, sem0, sem1, sem2, sem3):
    wid = lax.axis_index("s") * _NC + lax.axis_index("c")
    base = wid * _BPW
    pltpu.sync_copy(user_hbm.at[pl.ds(base, _BPW)], uidx_v.at[pl.ds(0, _BPW)])
    pltpu.sync_copy(item_hbm.at[pl.ds(base, _BPW)], iidx_v.at[pl.ds(0, _BPW)])
    half0 = lax.iota(jnp.int32, 16)
    half1 = half0 + 16
    sems = (sem0, sem1, sem2, sem3)
    lrow = wid * (_BPW // 128) * _EMBED

    def one_table(idx_v, tab_hbm, out_hbm):
        def fire(vec, k):
            s = vec[k]
            off = pl.multiple_of((s >> 7) * 128, 128)
            pltpu.async_copy(tab_hbm.at[:, pl.ds(off, 128)],
                             buf.at[pl.ds(k * _EMBED, _EMBED), :],
                             sems[k // 4])

        def extract(vec, e0, k):
            s = vec[k]
            c = jnp.full((16,), s & 127, jnp.int32)
            e = e0 + k
            col = jnp.full((16,), e & 127, jnp.int32)
            krow = (e // 128) * _EMBED
            for half in (half0, half1):
                v = plsc.load_gather(buf, [half + k * _EMBED, c])
                plsc.store_scatter(stage, [half + krow, col], v)

        vec0 = idx_v[pl.ds(0, 16)]
        for k in range(16):
            fire(vec0, k)

        def step(j, carry):
            e0 = j * 16
            vec = idx_v[pl.ds(e0, 16)]
            vnx = idx_v[pl.ds(e0 + 16, 16)]
            last = j == (_BPW // 16 - 1)
            for bank in range(4):
                for _ in range(4):
                    pltpu.make_async_copy(tab_hbm.at[:, pl.ds(0, 128)],
                                          buf.at[pl.ds(0, _EMBED), :],
                                          sems[bank]).wait()
                for k in range(4 * bank, 4 * bank + 4):
                    extract(vec, e0, k)

                @pl.when(jnp.logical_not(last))
                def _():
                    for k in range(4 * bank, 4 * bank + 4):
                        fire(vnx, k)
            return carry

        lax.fori_loop(0, _BPW // 16, step, 0)
        pltpu.sync_copy(stage, out_hbm.at[pl.ds(lrow, (_BPW // 128) * _EMBED), :])

    one_table(uidx_v, utab_hbm, uout_hbm)
    one_table(iidx_v, itab_hbm, iout_hbm)


_sc_gather = functools.partial(
    pl.kernel,
    mesh=plsc.VectorSubcoreMesh(core_axis_name="c", subcore_axis_name="s"),
    out_type=(
        jax.ShapeDtypeStruct((_LROWS, 128), jnp.float32),
        jax.ShapeDtypeStruct((_LROWS, 128), jnp.float32),
    ),
    scratch_types=[
        pltpu.VMEM((_BPW + 16,), jnp.int32),
        pltpu.VMEM((_BPW + 16,), jnp.int32),
        pltpu.VMEM((16 * _EMBED, 128), jnp.float32),
        pltpu.VMEM(((_BPW // 128) * _EMBED, 128), jnp.float32),
        pltpu.SemaphoreType.DMA,
        pltpu.SemaphoreType.DMA,
        pltpu.SemaphoreType.DMA,
        pltpu.SemaphoreType.DMA,
    ],
    compiler_params=pltpu.CompilerParams(needs_layout_passes=False),
)(_gather_body)


def _mlp_body(u_ref, i_ref, w1u_ref, w1i_ref, b1_ref, w2_ref, b2_ref,
              w3_ref, b3_ref, w4_ref, b4_ref, o_ref):
    x = jnp.dot(w1u_ref[...], u_ref[...], preferred_element_type=jnp.float32)
    x = x + jnp.dot(w1i_ref[...], i_ref[...], preferred_element_type=jnp.float32)
    x = jnp.maximum(x + b1_ref[...], 0.0)
    x = jnp.maximum(jnp.dot(w2_ref[...], x, preferred_element_type=jnp.float32)
                    + b2_ref[...], 0.0)
    x = jnp.maximum(jnp.dot(w3_ref[...], x, preferred_element_type=jnp.float32)
                    + b3_ref[...], 0.0)
    x = jnp.dot(w4_ref[...], x, preferred_element_type=jnp.float32) + b4_ref[...]
    o_ref[...] = jax.nn.sigmoid(x)


_GB = 16
_BR = _GB * _EMBED


def _mlp(u_l, i_l, w1u, w1i, b1, w2, b2, w3, b3, w4, b4):
    full = lambda shape: pl.BlockSpec(shape, lambda i: (0, 0))
    return pl.pallas_call(
        _mlp_body,
        grid=(_LROWS // _BR,),
        in_specs=[
            pl.BlockSpec((_BR, 128), lambda i: (i, 0)),
            pl.BlockSpec((_BR, 128), lambda i: (i, 0)),
            full(( _GB * 64, _BR)), full((_GB * 64, _BR)), full((_GB * 64, 1)),
            full((_GB * 32, _GB * 64)), full((_GB * 32, 1)),
            full((_GB * 16, _GB * 32)), full((_GB * 16, 1)),
            full((_GB, _GB * 16)), full((1, 1)),
        ],
        out_specs=pl.BlockSpec((_GB, 128), lambda i: (i, 0)),
        out_shape=jax.ShapeDtypeStruct((_HB // 128, 128), jnp.float32),
    )(u_l, i_l, w1u, w1i, b1, w2, b2, w3, b3, w4, b4)


def kernel(user, item, user_table, item_table, W1, b1, W2, b2, W3, b3, W4, b4):
    user = user.astype(jnp.int32)
    item = item.astype(jnp.int32)
    ut = user_table.T
    it = item_table.T
    eye = jnp.eye(_GB, dtype=jnp.float32)
    bd = lambda w: jnp.kron(eye, w)
    tile_b = lambda b: jnp.tile(b, (_GB,)).reshape(-1, 1)
    ws = (bd(W1[:_EMBED].T), bd(W1[_EMBED:].T), tile_b(b1),
          bd(W2.T), tile_b(b2), bd(W3.T), tile_b(b3),
          bd(W4.T), b4.reshape(1, 1))
    ys = []
    for h in range(2):
        u_l, i_l = _sc_gather(user[h * _HB:(h + 1) * _HB],
                              item[h * _HB:(h + 1) * _HB], ut, it)
        ys.append(_mlp(u_l, i_l, *ws).reshape(_HB, 1))
    return jnp.concatenate(ys, axis=0)


# speedup vs baseline: 1.0148x; 1.0148x over previous
import functools

import jax
import jax.numpy as jnp
from jax import lax
from jax.experimental import pallas as pl
from jax.experimental.pallas import tpu as pltpu
from jax.experimental.pallas import tpu_sc as plsc

_EMBED = 32
_BATCH = 16384
_NC = 2
_NS = 16
_NW = _NC * _NS
_BPW = _BATCH // _NW
_K = 8
_NG = _BPW // _K
_LROWS = (_BATCH // 128) * _EMBED


def _gather_body(user_hbm, item_hbm, utab_hbm, itab_hbm, uout_hbm, iout_hbm,
                 uidx_v, iidx_v, buf, stage, sem0, sem1, sem2, sem3):
    wid = lax.axis_index("s") * _NC + lax.axis_index("c")
    base = wid * _BPW
    pltpu.sync_copy(user_hbm.at[pl.ds(base, _BPW)], uidx_v.at[pl.ds(0, _BPW)])
    pltpu.sync_copy(item_hbm.at[pl.ds(base, _BPW)], iidx_v.at[pl.ds(0, _BPW)])
    half0 = lax.iota(jnp.int32, 16)
    half1 = half0 + 16
    sems = (sem0, sem1, sem2, sem3)
    lrow = wid * (_BPW // 128) * _EMBED

    def one_table(idx_v, tab_hbm, out_hbm):
        def fire(vec, k):
            s = vec[k]
            off = pl.multiple_of((s >> 7) * 128, 128)
            pltpu.async_copy(tab_hbm.at[:, pl.ds(off, 128)],
                             buf.at[pl.ds(k * _EMBED, _EMBED), :],
                             sems[k // 4])

        def extract(vec, e0, k):
            s = vec[k]
            c = jnp.full((16,), s & 127, jnp.int32)
            e = e0 + k
            col = jnp.full((16,), e & 127, jnp.int32)
            krow = (e // 128) * _EMBED
            for half in (half0, half1):
                v = plsc.load_gather(buf, [half + k * _EMBED, c])
                plsc.store_scatter(stage, [half + krow, col], v)

        vec0 = idx_v[pl.ds(0, 16)]
        for k in range(16):
            fire(vec0, k)

        def step(j, carry):
            e0 = j * 16
            vec = idx_v[pl.ds(e0, 16)]
            vnx = idx_v[pl.ds(e0 + 16, 16)]
            last = j == (_BPW // 16 - 1)
            for bank in range(4):
                for _ in range(4):
                    pltpu.make_async_copy(tab_hbm.at[:, pl.ds(0, 128)],
                                          buf.at[pl.ds(0, _EMBED), :],
                                          sems[bank]).wait()
                for k in range(4 * bank, 4 * bank + 4):
                    extract(vec, e0, k)

                @pl.when(jnp.logical_not(last))
                def _():
                    for k in range(4 * bank, 4 * bank + 4):
                        fire(vnx, k)
            return carry

        lax.fori_loop(0, _BPW // 16, step, 0)
        pltpu.sync_copy(stage, out_hbm.at[pl.ds(lrow, (_BPW // 128) * _EMBED), :])

    one_table(uidx_v, utab_hbm, uout_hbm)
    one_table(iidx_v, itab_hbm, iout_hbm)


_sc_gather = functools.partial(
    pl.kernel,
    mesh=plsc.VectorSubcoreMesh(core_axis_name="c", subcore_axis_name="s"),
    out_type=(
        jax.ShapeDtypeStruct((_LROWS, 128), jnp.float32),
        jax.ShapeDtypeStruct((_LROWS, 128), jnp.float32),
    ),
    scratch_types=[
        pltpu.VMEM((_BPW + 16,), jnp.int32),
        pltpu.VMEM((_BPW + 16,), jnp.int32),
        pltpu.VMEM((16 * _EMBED, 128), jnp.float32),
        pltpu.VMEM(((_BPW // 128) * _EMBED, 128), jnp.float32),
        pltpu.SemaphoreType.DMA,
        pltpu.SemaphoreType.DMA,
        pltpu.SemaphoreType.DMA,
        pltpu.SemaphoreType.DMA,
    ],
    compiler_params=pltpu.CompilerParams(needs_layout_passes=False),
)(_gather_body)


def _mlp_body(u_ref, i_ref, w1u_ref, w1i_ref, b1_ref, w2_ref, b2_ref,
              w3_ref, b3_ref, w4_ref, b4_ref, o_ref):
    x = jnp.dot(w1u_ref[...], u_ref[...], preferred_element_type=jnp.float32)
    x = x + jnp.dot(w1i_ref[...], i_ref[...], preferred_element_type=jnp.float32)
    x = jnp.maximum(x + b1_ref[...], 0.0)
    x = jnp.maximum(jnp.dot(w2_ref[...], x, preferred_element_type=jnp.float32)
                    + b2_ref[...], 0.0)
    x = jnp.maximum(jnp.dot(w3_ref[...], x, preferred_element_type=jnp.float32)
                    + b3_ref[...], 0.0)
    x = jnp.dot(w4_ref[...], x, preferred_element_type=jnp.float32) + b4_ref[...]
    o_ref[...] = jax.nn.sigmoid(x)


_GB = 16
_BR = _GB * _EMBED


def _mlp(u_l, i_l, w1u, w1i, b1, w2, b2, w3, b3, w4, b4):
    full = lambda shape: pl.BlockSpec(shape, lambda i: (0, 0))
    return pl.pallas_call(
        _mlp_body,
        grid=(_LROWS // _BR,),
        in_specs=[
            pl.BlockSpec((_BR, 128), lambda i: (i, 0)),
            pl.BlockSpec((_BR, 128), lambda i: (i, 0)),
            full(( _GB * 64, _BR)), full((_GB * 64, _BR)), full((_GB * 64, 1)),
            full((_GB * 32, _GB * 64)), full((_GB * 32, 1)),
            full((_GB * 16, _GB * 32)), full((_GB * 16, 1)),
            full((_GB, _GB * 16)), full((1, 1)),
        ],
        out_specs=pl.BlockSpec((_GB, 128), lambda i: (i, 0)),
        out_shape=jax.ShapeDtypeStruct((_BATCH // 128, 128), jnp.float32),
    )(u_l, i_l, w1u, w1i, b1, w2, b2, w3, b3, w4, b4)


def kernel(user, item, user_table, item_table, W1, b1, W2, b2, W3, b3, W4, b4):
    user = user.astype(jnp.int32)
    item = item.astype(jnp.int32)
    u_l, i_l = _sc_gather(user, item, user_table.T, item_table.T)
    eye = jnp.eye(_GB, dtype=jnp.float32)
    bd = lambda w: jnp.kron(eye, w)
    tile_b = lambda b: jnp.tile(b, (_GB,)).reshape(-1, 1)
    y = _mlp(
        u_l, i_l,
        bd(W1[:_EMBED].T), bd(W1[_EMBED:].T), tile_b(b1),
        bd(W2.T), tile_b(b2),
        bd(W3.T), tile_b(b3),
        bd(W4.T), b4.reshape(1, 1),
    )
    return y.reshape(_BATCH, 1)
